# Initial kernel scaffold; baseline (speedup 1.0000x reference)
import jax
import jax.numpy as jnp
from jax.experimental import pallas as pl


def kernel(x, table):
    raise NotImplementedError("write your pallas kernel here")


# speedup vs baseline: 2.9140x; 2.9140x over previous
import functools

import jax
import jax.numpy as jnp
from jax import lax
from jax.experimental import pallas as pl
from jax.experimental.pallas import tpu as pltpu
from jax.experimental.pallas import tpu_sc as plsc

_B, _S, _D = 4, 4096, 1024
_NC, _NS = 2, 16
_NW = _NC * _NS
_ROWS = _S // _NW
_CHUNK = 32
_NCHUNK = _ROWS // _CHUNK

_mesh = plsc.VectorSubcoreMesh(core_axis_name="c", subcore_axis_name="s")


@functools.partial(
    pl.kernel,
    mesh=_mesh,
    out_type=jax.ShapeDtypeStruct((_B * _S, _D), jnp.float32),
    scratch_types=[
        pltpu.VMEM((_CHUNK, _D), jnp.float32),
        pltpu.SemaphoreType.DMA,
        pltpu.SemaphoreType.DMA,
    ],
)
def _bcast_rows(table_hbm, out_hbm, buf, sem_in, sem_out):
    wid = lax.axis_index("s") * _NC + lax.axis_index("c")
    base = wid * _ROWS

    def chunk_body(i, carry):
        row0 = base + i * _CHUNK
        pltpu.async_copy(table_hbm.at[pl.ds(row0, _CHUNK)], buf, sem_in).wait()
        copies = [
            pltpu.async_copy(buf, out_hbm.at[pl.ds(b * _S + row0, _CHUNK)], sem_out)
            for b in range(_B)
        ]
        for cp in copies:
            cp.wait()
        return carry

    lax.fori_loop(0, _NCHUNK, chunk_body, 0)


def kernel(x, table):
    del x
    out = _bcast_rows(table)
    return out.reshape(_B, _S, _D)


# speedup vs baseline: 2.9601x; 1.0158x over previous
import functools

import jax
import jax.numpy as jnp
from jax import lax
from jax.experimental import pallas as pl
from jax.experimental.pallas import tpu as pltpu
from jax.experimental.pallas import tpu_sc as plsc

_B, _S, _D = 4, 4096, 1024
_NC, _NS = 2, 16
_NW = _NC * _NS
_ROWS = _S // _NW
_CHUNK = 32
_NCHUNK = _ROWS // _CHUNK
_NBUF = 3

_mesh = plsc.VectorSubcoreMesh(core_axis_name="c", subcore_axis_name="s")


@functools.partial(
    pl.kernel,
    mesh=_mesh,
    out_type=jax.ShapeDtypeStruct((_B * _S, _D), jnp.float32),
    scratch_types=[
        pltpu.VMEM((_NBUF, _CHUNK, _D), jnp.float32),
        pltpu.SemaphoreType.DMA((_NBUF,)),
        pltpu.SemaphoreType.DMA((_NBUF,)),
    ],
)
def _bcast_rows(table_hbm, out_hbm, buf, in_sems, out_sems):
    wid = lax.axis_index("s") * _NC + lax.axis_index("c")
    base = wid * _ROWS

    reads = [None] * _NCHUNK
    writes = [None] * _NCHUNK

    def start_read(i):
        s = i % _NBUF
        reads[i] = pltpu.async_copy(
            table_hbm.at[pl.ds(base + i * _CHUNK, _CHUNK)], buf.at[s], in_sems.at[s]
        )

    def fire_writes(i):
        s = i % _NBUF
        writes[i] = [
            pltpu.async_copy(
                buf.at[s],
                out_hbm.at[pl.ds(b * _S + base + i * _CHUNK, _CHUNK)],
                out_sems.at[s],
            )
            for b in range(_B)
        ]

    for i in range(min(_NBUF, _NCHUNK)):
        start_read(i)
    for i in range(_NCHUNK):
        reads[i].wait()
        fire_writes(i)
        nxt = i + 1
        if _NBUF <= nxt < _NCHUNK:
            for cp in writes[nxt - _NBUF]:
                cp.wait()
            start_read(nxt)
    for i in range(max(0, _NCHUNK - _NBUF), _NCHUNK):
        for cp in writes[i]:
            cp.wait()


def kernel(x, table):
    del x
    out = _bcast_rows(table)
    return out.reshape(_B, _S, _D)
